# baseline (speedup 1.0000x reference)
import jax
import jax.numpy as jnp
import numpy as np
from jax import lax
from jax.experimental import pallas as pl
from jax.experimental.pallas import tpu as pltpu
from jax.experimental.pallas import tpu_sc as plsc

DIM = 384
N_VIEWS = 8
PATCH_GRID = 16
TABLE_ROWS = N_VIEWS * PATCH_GRID * PATCH_GRID
SENTINEL = TABLE_ROWS

P_ROWS = 8
P_COLS = 256
P_BLK = P_ROWS * P_COLS
N_OUT = 100000
N_PAD = 100352

NW = 32
B_PER_W = N_PAD // NW
CH = 56
NCH = B_PER_W // CH

_CHUNK_ROWS = np.minimum(
    (np.arange(NW) * B_PER_W)[:, None, None]
    + np.minimum((np.arange(NCH) * CH)[None, :, None],
                 np.minimum(N_OUT - CH - np.arange(NW) * B_PER_W,
                            B_PER_W - CH)[:, None, None])
    + np.arange(CH)[None, None, :], N_OUT - 1).astype(np.int32).reshape(-1)


def _dino_matmul_kernel(x_ref, w_ref, o_ref):
    o_ref[...] = jnp.dot(x_ref[...].astype(jnp.bfloat16),
                         w_ref[...].astype(jnp.bfloat16),
                         preferred_element_type=jnp.float32)


def _rb(t):
    return t.astype(jnp.bfloat16).astype(jnp.float32)


def _index_kernel(par_ref, x_ref, y_ref, z_ref, b_ref, o_ref):
    x = _rb(x_ref[...])
    y = _rb(y_ref[...])
    z = _rb(z_ref[...])
    bidx = b_ref[...]
    idx = jnp.full(x.shape, SENTINEL, jnp.int32)
    for v8 in range(N_VIEWS):
        e = [_rb(par_ref[v8, i]) for i in range(12)]
        k = [_rb(par_ref[v8, 12 + i]) for i in range(9)]
        xc = e[0] * x + e[1] * y + e[2] * z + e[3]
        yc = e[4] * x + e[5] * y + e[6] * z + e[7]
        zc = e[8] * x + e[9] * y + e[10] * z + e[11]
        xcb, ycb, zcb = _rb(xc), _rb(yc), _rb(zc)
        xi = k[0] * xcb + k[1] * ycb + k[2] * zcb
        yi = k[3] * xcb + k[4] * ycb + k[5] * zcb
        zi = k[6] * xcb + k[7] * ycb + k[8] * zcb
        u = xi / zi
        v = yi / zi
        valid = ((zc > 0.1) & (u >= 0.0) & (u < 224.0)
                 & (v >= 0.0) & (v < 224.0) & (bidx == (v8 // 4)))
        up = jnp.clip((u / 14.0).astype(jnp.int32), 0, PATCH_GRID - 1)
        vp = jnp.clip((v / 14.0).astype(jnp.int32), 0, PATCH_GRID - 1)
        cand = v8 * (PATCH_GRID * PATCH_GRID) + vp * PATCH_GRID + up
        idx = jnp.where(valid, cand, idx)
    o_ref[...] = idx


def _gather_body(table_hbm, idx_hbm, out_hbm, idx_v, buf0, buf1,
                 sem0, sem1):
    wid = lax.axis_index("s") * 2 + lax.axis_index("c")
    base = wid * B_PER_W
    local_max = jnp.minimum(N_OUT - CH - base, B_PER_W - CH)

    pltpu.sync_copy(idx_hbm.at[wid], idx_v)

    def start(c, buf, sem):
        pltpu.async_copy(table_hbm.at[idx_v.at[c]], buf, sem)

    def wait(buf, sem):
        pltpu.make_async_copy(table_hbm.at[pl.ds(0, CH)], buf, sem).wait()

    def store(c, buf):
        local = jnp.minimum(c * CH, local_max)
        pltpu.sync_copy(buf, out_hbm.at[pl.ds(base + local, CH)])

    start(0, buf0, sem0)

    def body(i, carry):
        c0 = 2 * i
        start(c0 + 1, buf1, sem1)
        wait(buf0, sem0)
        store(c0, buf0)

        @pl.when(i < NCH // 2 - 1)
        def _():
            start(c0 + 2, buf0, sem0)

        wait(buf1, sem1)
        store(c0 + 1, buf1)
        return carry

    lax.fori_loop(0, NCH // 2, body, 0)


def kernel(points, batch_idx, imgs, intrinsics, extrinsics, W_dino):
    b, v, c, h, w = imgs.shape
    x = imgs.reshape(b * v, c, PATCH_GRID, 14, PATCH_GRID, 14)
    x = x.transpose(0, 2, 4, 1, 3, 5).reshape(b * v * PATCH_GRID * PATCH_GRID,
                                              c * 14 * 14)
    table = pl.pallas_call(
        _dino_matmul_kernel,
        out_shape=jax.ShapeDtypeStruct((TABLE_ROWS, DIM), jnp.float32),
    )(x, W_dino)
    table_pad = jnp.concatenate(
        [table, jnp.zeros((8, DIM), jnp.float32)], axis=0)

    n = points.shape[0]
    pad = N_PAD - n
    pts = jnp.pad(points, ((0, pad), (0, 0)))
    bi = jnp.pad(batch_idx, (0, pad))
    xs = pts[:, 0].reshape(-1, P_COLS)
    ys = pts[:, 1].reshape(-1, P_COLS)
    zs = pts[:, 2].reshape(-1, P_COLS)
    bi2 = bi.reshape(-1, P_COLS)
    params = jnp.concatenate(
        [extrinsics.reshape(N_VIEWS, 12), intrinsics.reshape(N_VIEWS, 9),
         jnp.zeros((N_VIEWS, 3), jnp.float32)], axis=1)

    grid = N_PAD // P_BLK
    blk = pl.BlockSpec((P_ROWS, P_COLS), lambda i: (i, 0))
    idx = pl.pallas_call(
        _index_kernel,
        grid=(grid,),
        in_specs=[pl.BlockSpec(memory_space=pltpu.SMEM), blk, blk, blk, blk],
        out_specs=blk,
        out_shape=jax.ShapeDtypeStruct((N_PAD // P_COLS, P_COLS), jnp.int32),
    )(params, xs, ys, zs, bi2)

    idx3 = jnp.take(idx.reshape(-1), _CHUNK_ROWS, axis=0).reshape(NW, NCH, CH)

    mesh = plsc.VectorSubcoreMesh(core_axis_name="c", subcore_axis_name="s")
    out = pl.kernel(
        _gather_body,
        out_type=jax.ShapeDtypeStruct((N_OUT, DIM), jnp.float32),
        mesh=mesh,
        compiler_params=pltpu.CompilerParams(use_tc_tiling_on_sc=False),
        scratch_types=[
            pltpu.VMEM((NCH, CH), jnp.int32),
            pltpu.VMEM((CH, DIM), jnp.float32),
            pltpu.VMEM((CH, DIM), jnp.float32),
            pltpu.SemaphoreType.DMA,
            pltpu.SemaphoreType.DMA,
        ],
    )(table_pad, idx3)
    return out


# speedup vs baseline: 1.0524x; 1.0524x over previous
import jax
import jax.numpy as jnp
import numpy as np
from jax import lax
from jax.experimental import pallas as pl
from jax.experimental.pallas import tpu as pltpu
from jax.experimental.pallas import tpu_sc as plsc

DIM = 384
N_VIEWS = 8
PATCH_GRID = 16
TABLE_ROWS = N_VIEWS * PATCH_GRID * PATCH_GRID
SENTINEL = TABLE_ROWS

P_ROWS = 8
P_COLS = 256
P_BLK = P_ROWS * P_COLS
N_OUT = 100000
N_PAD = 100352

NW = 32
B_PER_W = N_PAD // NW
RCH = 16
NCH = B_PER_W // RCH


def _dino_matmul_kernel(x_ref, w_ref, o_ref):
    o_ref[...] = jnp.dot(x_ref[...].astype(jnp.bfloat16),
                         w_ref[...].astype(jnp.bfloat16),
                         preferred_element_type=jnp.float32)


def _rb(t):
    return t.astype(jnp.bfloat16).astype(jnp.float32)


def _index_kernel(par_ref, x_ref, y_ref, z_ref, b_ref, o_ref):
    x = _rb(x_ref[...])
    y = _rb(y_ref[...])
    z = _rb(z_ref[...])
    bidx = b_ref[...]
    idx = jnp.full(x.shape, SENTINEL, jnp.int32)
    for v8 in range(N_VIEWS):
        e = [_rb(par_ref[v8, i]) for i in range(12)]
        k = [_rb(par_ref[v8, 12 + i]) for i in range(9)]
        xc = e[0] * x + e[1] * y + e[2] * z + e[3]
        yc = e[4] * x + e[5] * y + e[6] * z + e[7]
        zc = e[8] * x + e[9] * y + e[10] * z + e[11]
        xcb, ycb, zcb = _rb(xc), _rb(yc), _rb(zc)
        xi = k[0] * xcb + k[1] * ycb + k[2] * zcb
        yi = k[3] * xcb + k[4] * ycb + k[5] * zcb
        zi = k[6] * xcb + k[7] * ycb + k[8] * zcb
        u = xi / zi
        v = yi / zi
        valid = ((zc > 0.1) & (u >= 0.0) & (u < 224.0)
                 & (v >= 0.0) & (v < 224.0) & (bidx == (v8 // 4)))
        up = jnp.clip((u / 14.0).astype(jnp.int32), 0, PATCH_GRID - 1)
        vp = jnp.clip((v / 14.0).astype(jnp.int32), 0, PATCH_GRID - 1)
        cand = v8 * (PATCH_GRID * PATCH_GRID) + vp * PATCH_GRID + up
        idx = jnp.where(valid, cand, idx)
    o_ref[...] = idx


def _gather_body(table_hbm, idx_hbm, out_hbm, idx_v, buf0, buf1,
                 sem0, sem1):
    wid = lax.axis_index("s") * 2 + lax.axis_index("c")
    base = wid * B_PER_W
    local_max = jnp.minimum(N_OUT - RCH - base, B_PER_W - RCH)

    pltpu.sync_copy(idx_hbm.at[pl.ds(base, B_PER_W)], idx_v)

    def start(c, buf, sem):
        local = jnp.minimum(c * RCH, local_max)
        vec = idx_v[pl.ds(local, RCH)]
        for j in range(RCH):
            pltpu.async_copy(table_hbm.at[pl.ds(vec[j], 1)],
                             buf.at[pl.ds(j, 1)], sem)

    def wait(buf, sem):
        pltpu.make_async_copy(table_hbm.at[pl.ds(0, RCH)], buf, sem).wait()

    def store(c, buf):
        local = jnp.minimum(c * RCH, local_max)
        pltpu.sync_copy(buf, out_hbm.at[pl.ds(base + local, RCH)])

    start(0, buf0, sem0)

    def body(i, carry):
        c0 = 2 * i
        start(c0 + 1, buf1, sem1)
        wait(buf0, sem0)
        store(c0, buf0)

        @pl.when(i < NCH // 2 - 1)
        def _():
            start(c0 + 2, buf0, sem0)

        wait(buf1, sem1)
        store(c0 + 1, buf1)
        return carry

    lax.fori_loop(0, NCH // 2, body, 0)


def kernel(points, batch_idx, imgs, intrinsics, extrinsics, W_dino):
    b, v, c, h, w = imgs.shape
    x = imgs.reshape(b * v, c, PATCH_GRID, 14, PATCH_GRID, 14)
    x = x.transpose(0, 2, 4, 1, 3, 5).reshape(b * v * PATCH_GRID * PATCH_GRID,
                                              c * 14 * 14)
    table = pl.pallas_call(
        _dino_matmul_kernel,
        out_shape=jax.ShapeDtypeStruct((TABLE_ROWS, DIM), jnp.float32),
    )(x, W_dino)
    table_pad = jnp.concatenate(
        [table, jnp.zeros((8, DIM), jnp.float32)], axis=0)

    n = points.shape[0]
    pad = N_PAD - n
    pts = jnp.pad(points, ((0, pad), (0, 0)))
    bi = jnp.pad(batch_idx, (0, pad))
    xs = pts[:, 0].reshape(-1, P_COLS)
    ys = pts[:, 1].reshape(-1, P_COLS)
    zs = pts[:, 2].reshape(-1, P_COLS)
    bi2 = bi.reshape(-1, P_COLS)
    params = jnp.concatenate(
        [extrinsics.reshape(N_VIEWS, 12), intrinsics.reshape(N_VIEWS, 9),
         jnp.zeros((N_VIEWS, 3), jnp.float32)], axis=1)

    grid = N_PAD // P_BLK
    blk = pl.BlockSpec((P_ROWS, P_COLS), lambda i: (i, 0))
    idx = pl.pallas_call(
        _index_kernel,
        grid=(grid,),
        in_specs=[pl.BlockSpec(memory_space=pltpu.SMEM), blk, blk, blk, blk],
        out_specs=blk,
        out_shape=jax.ShapeDtypeStruct((N_PAD // P_COLS, P_COLS), jnp.int32),
    )(params, xs, ys, zs, bi2)

    mesh = plsc.VectorSubcoreMesh(core_axis_name="c", subcore_axis_name="s")
    out = pl.kernel(
        _gather_body,
        out_type=jax.ShapeDtypeStruct((N_OUT, DIM), jnp.float32),
        mesh=mesh,
        scratch_types=[
            pltpu.VMEM((B_PER_W,), jnp.int32),
            pltpu.VMEM((RCH, DIM), jnp.float32),
            pltpu.VMEM((RCH, DIM), jnp.float32),
            pltpu.SemaphoreType.DMA,
            pltpu.SemaphoreType.DMA,
        ],
    )(table_pad, idx.reshape(-1))
    return out


# speedup vs baseline: 1.6424x; 1.5606x over previous
import jax
import jax.numpy as jnp
from jax import lax
from jax.experimental import pallas as pl
from jax.experimental.pallas import tpu as pltpu
from jax.experimental.pallas import tpu_sc as plsc

DIM = 384
N_VIEWS = 8
PATCH_GRID = 16
TABLE_ROWS = N_VIEWS * PATCH_GRID * PATCH_GRID
TPAD = TABLE_ROWS + 8
SENTINEL = TABLE_ROWS

P_ROWS = 8
P_COLS = 256
P_BLK = P_ROWS * P_COLS
N_OUT = 100000
N_PAD = 100352

NCG = 8
CW = 48
NRG = 4
R_PER_G = N_PAD // NRG
P = 256
NCHK = R_PER_G // P


def _dino_matmul_kernel(x_ref, w_ref, o_ref):
    o_ref[...] = jnp.dot(x_ref[...].astype(jnp.bfloat16),
                         w_ref[...].astype(jnp.bfloat16),
                         preferred_element_type=jnp.float32)


def _rb(t):
    return t.astype(jnp.bfloat16).astype(jnp.float32)


def _index_kernel(par_ref, x_ref, y_ref, z_ref, b_ref, o_ref):
    x = _rb(x_ref[...])
    y = _rb(y_ref[...])
    z = _rb(z_ref[...])
    bidx = b_ref[...]
    idx = jnp.full(x.shape, SENTINEL * CW, jnp.int32)
    for v8 in range(N_VIEWS):
        e = [_rb(par_ref[v8, i]) for i in range(12)]
        k = [_rb(par_ref[v8, 12 + i]) for i in range(9)]
        xc = e[0] * x + e[1] * y + e[2] * z + e[3]
        yc = e[4] * x + e[5] * y + e[6] * z + e[7]
        zc = e[8] * x + e[9] * y + e[10] * z + e[11]
        xcb, ycb, zcb = _rb(xc), _rb(yc), _rb(zc)
        xi = k[0] * xcb + k[1] * ycb + k[2] * zcb
        yi = k[3] * xcb + k[4] * ycb + k[5] * zcb
        zi = k[6] * xcb + k[7] * ycb + k[8] * zcb
        u = xi / zi
        v = yi / zi
        valid = ((zc > 0.1) & (u >= 0.0) & (u < 224.0)
                 & (v >= 0.0) & (v < 224.0) & (bidx == (v8 // 4)))
        up = jnp.clip((u / 14.0).astype(jnp.int32), 0, PATCH_GRID - 1)
        vp = jnp.clip((v / 14.0).astype(jnp.int32), 0, PATCH_GRID - 1)
        cand = (v8 * (PATCH_GRID * PATCH_GRID) + vp * PATCH_GRID + up) * CW
        idx = jnp.where(valid, cand, idx)
    o_ref[...] = idx


def _gather_body(tblg_hbm, idx_hbm, out_hbm, tbl_v, ibuf, buf0, buf1,
                 isem0, isem1, ssem0, ssem1):
    wid = lax.axis_index("s") * 2 + lax.axis_index("c")
    cg = wid // NRG
    rg = wid % NRG
    rbase = rg * R_PER_G
    cbase = rg * NCHK
    tsz = TPAD * CW

    pltpu.sync_copy(tblg_hbm.at[pl.ds(cg * tsz, tsz)], tbl_v)

    siota = lax.iota(jnp.int32, 16) * CW

    def iload(c, par):
        sl = idx_hbm.at[pl.ds(cbase + c, 1)]
        pltpu.async_copy(sl, ibuf.at[pl.ds(par, 1)], (isem0, isem1)[par])

    def iwait(par):
        pltpu.make_async_copy(idx_hbm.at[pl.ds(0, 1)],
                              ibuf.at[pl.ds(par, 1)],
                              (isem0, isem1)[par]).wait()

    def compute(par, buf):
        def grp(q, carry):
            row48 = ibuf[par, pl.ds(q * 16, 16)]
            sb = siota + q * (16 * CW)
            for j in range(CW):
                vals = plsc.load_gather(tbl_v, [row48 + j])
                plsc.store_scatter(buf, [sb + j], vals)
            return carry
        lax.fori_loop(0, P // 16, grp, 0)

    def sstart(c, buf, ssem):
        dst = out_hbm.at[pl.ds((cg * N_PAD + rbase + c * P) * CW, P * CW)]
        pltpu.async_copy(buf, dst, ssem)

    def swait(buf, ssem):
        pltpu.make_async_copy(
            buf, out_hbm.at[pl.ds(0, P * CW)], ssem).wait()

    iload(0, 0)
    iload(1, 1)

    def body(i, carry):
        c0 = 2 * i

        @pl.when(i > 0)
        def _():
            swait(buf0, ssem0)

        iwait(0)
        compute(0, buf0)
        sstart(c0, buf0, ssem0)

        @pl.when(i < NCHK // 2 - 1)
        def _():
            iload(c0 + 2, 0)

        @pl.when(i > 0)
        def _():
            swait(buf1, ssem1)

        iwait(1)
        compute(1, buf1)
        sstart(c0 + 1, buf1, ssem1)

        @pl.when(i < NCHK // 2 - 1)
        def _():
            iload(c0 + 3, 1)

        return carry

    lax.fori_loop(0, NCHK // 2, body, 0)
    swait(buf0, ssem0)
    swait(buf1, ssem1)


def kernel(points, batch_idx, imgs, intrinsics, extrinsics, W_dino):
    b, v, c, h, w = imgs.shape
    x = imgs.reshape(b * v, c, PATCH_GRID, 14, PATCH_GRID, 14)
    x = x.transpose(0, 2, 4, 1, 3, 5).reshape(b * v * PATCH_GRID * PATCH_GRID,
                                              c * 14 * 14)
    table = pl.pallas_call(
        _dino_matmul_kernel,
        out_shape=jax.ShapeDtypeStruct((TABLE_ROWS, DIM), jnp.float32),
    )(x, W_dino)
    table_pad = jnp.concatenate(
        [table, jnp.zeros((TPAD - TABLE_ROWS, DIM), jnp.float32)], axis=0)
    tblg = table_pad.reshape(TPAD, NCG, CW).transpose(1, 0, 2).reshape(-1)

    n = points.shape[0]
    pad = N_PAD - n
    pts = jnp.pad(points, ((0, pad), (0, 0)))
    bi = jnp.pad(batch_idx, (0, pad))
    xs = pts[:, 0].reshape(-1, P_COLS)
    ys = pts[:, 1].reshape(-1, P_COLS)
    zs = pts[:, 2].reshape(-1, P_COLS)
    bi2 = bi.reshape(-1, P_COLS)
    params = jnp.concatenate(
        [extrinsics.reshape(N_VIEWS, 12), intrinsics.reshape(N_VIEWS, 9),
         jnp.zeros((N_VIEWS, 3), jnp.float32)], axis=1)

    grid = N_PAD // P_BLK
    blk = pl.BlockSpec((P_ROWS, P_COLS), lambda i: (i, 0))
    idx = pl.pallas_call(
        _index_kernel,
        grid=(grid,),
        in_specs=[pl.BlockSpec(memory_space=pltpu.SMEM), blk, blk, blk, blk],
        out_specs=blk,
        out_shape=jax.ShapeDtypeStruct((N_PAD // P_COLS, P_COLS), jnp.int32),
    )(params, xs, ys, zs, bi2)

    mesh = plsc.VectorSubcoreMesh(core_axis_name="c", subcore_axis_name="s")
    tmp = pl.kernel(
        _gather_body,
        out_type=jax.ShapeDtypeStruct((NCG * N_PAD * CW,), jnp.float32),
        mesh=mesh,
        compiler_params=pltpu.CompilerParams(needs_layout_passes=False),
        scratch_types=[
            pltpu.VMEM((TPAD * CW,), jnp.float32),
            pltpu.VMEM((2, P), jnp.int32),
            pltpu.VMEM((P * CW,), jnp.float32),
            pltpu.VMEM((P * CW,), jnp.float32),
            pltpu.SemaphoreType.DMA,
            pltpu.SemaphoreType.DMA,
            pltpu.SemaphoreType.DMA,
            pltpu.SemaphoreType.DMA,
        ],
    )(tblg, idx.reshape(-1, P))
    return (tmp.reshape(NCG, N_PAD, CW).transpose(1, 0, 2)
            .reshape(N_PAD, DIM)[:n])


# speedup vs baseline: 1.9578x; 1.1920x over previous
import jax
import jax.numpy as jnp
from jax import lax
from jax.experimental import pallas as pl
from jax.experimental.pallas import tpu as pltpu
from jax.experimental.pallas import tpu_sc as plsc

DIM = 384
N_VIEWS = 8
PATCH_GRID = 16
TABLE_ROWS = N_VIEWS * PATCH_GRID * PATCH_GRID
TPAD = TABLE_ROWS + 8
SENTINEL = TABLE_ROWS

P_ROWS = 8
P_COLS = 256
P_BLK = P_ROWS * P_COLS
N_OUT = 100000
N_PAD = 100352

NCG = 8
CW = 48
NRG = 4
R_PER_G = N_PAD // NRG
P = 256
NCHK = R_PER_G // P


def _dino_matmul_kernel(x_ref, w_ref, o_ref):
    o_ref[...] = jnp.dot(x_ref[...].astype(jnp.bfloat16),
                         w_ref[...].astype(jnp.bfloat16),
                         preferred_element_type=jnp.float32)


def _rb(t):
    return t.astype(jnp.bfloat16).astype(jnp.float32)


def _index_kernel(par_ref, x_ref, y_ref, z_ref, b_ref, o_ref):
    x = _rb(x_ref[...])
    y = _rb(y_ref[...])
    z = _rb(z_ref[...])
    bidx = b_ref[...]
    idx = jnp.full(x.shape, SENTINEL * CW, jnp.int32)
    for v8 in range(N_VIEWS):
        e = [_rb(par_ref[v8, i]) for i in range(12)]
        k = [_rb(par_ref[v8, 12 + i]) for i in range(9)]
        xc = e[0] * x + e[1] * y + e[2] * z + e[3]
        yc = e[4] * x + e[5] * y + e[6] * z + e[7]
        zc = e[8] * x + e[9] * y + e[10] * z + e[11]
        xcb, ycb, zcb = _rb(xc), _rb(yc), _rb(zc)
        xi = k[0] * xcb + k[1] * ycb + k[2] * zcb
        yi = k[3] * xcb + k[4] * ycb + k[5] * zcb
        zi = k[6] * xcb + k[7] * ycb + k[8] * zcb
        u = xi / zi
        v = yi / zi
        valid = ((zc > 0.1) & (u >= 0.0) & (u < 224.0)
                 & (v >= 0.0) & (v < 224.0) & (bidx == (v8 // 4)))
        up = jnp.clip((u / 14.0).astype(jnp.int32), 0, PATCH_GRID - 1)
        vp = jnp.clip((v / 14.0).astype(jnp.int32), 0, PATCH_GRID - 1)
        cand = (v8 * (PATCH_GRID * PATCH_GRID) + vp * PATCH_GRID + up) * CW
        idx = jnp.where(valid, cand, idx)
    o_ref[...] = idx


def _gather_body(tblg_hbm, idx_hbm, out_hbm, tbl_v, ibuf, buf0, buf1,
                 isem0, isem1, ssem0, ssem1):
    wid = lax.axis_index("s") * 2 + lax.axis_index("c")
    cg = wid // NRG
    rg = wid % NRG
    rbase = rg * R_PER_G
    cbase = rg * NCHK
    tsz = TPAD * CW

    pltpu.sync_copy(tblg_hbm.at[pl.ds(cg * tsz, tsz)], tbl_v)

    siota = lax.iota(jnp.int32, 16) * CW

    def iload(c, par):
        sl = idx_hbm.at[pl.ds(cbase + c, 1)]
        pltpu.async_copy(sl, ibuf.at[pl.ds(par, 1)], (isem0, isem1)[par])

    def iwait(par):
        pltpu.make_async_copy(idx_hbm.at[pl.ds(0, 1)],
                              ibuf.at[pl.ds(par, 1)],
                              (isem0, isem1)[par]).wait()

    def compute(par, buf):
        def grp(q, carry):
            row48 = ibuf[par, pl.ds(q * 16, 16)]
            sb = siota + q * (16 * CW)
            for j0 in range(0, CW, 8):
                vals = [plsc.load_gather(tbl_v, [row48 + (j0 + t)])
                        for t in range(8)]
                for t in range(8):
                    plsc.store_scatter(buf, [sb + (j0 + t)], vals[t])
            return carry
        lax.fori_loop(0, P // 16, grp, 0)

    def sstart(c, buf, ssem):
        dst = out_hbm.at[pl.ds((cg * N_PAD + rbase + c * P) * CW, P * CW)]
        pltpu.async_copy(buf, dst, ssem)

    def swait(buf, ssem):
        pltpu.make_async_copy(
            buf, out_hbm.at[pl.ds(0, P * CW)], ssem).wait()

    iload(0, 0)
    iload(1, 1)

    def body(i, carry):
        c0 = 2 * i

        @pl.when(i > 0)
        def _():
            swait(buf0, ssem0)

        iwait(0)
        compute(0, buf0)
        sstart(c0, buf0, ssem0)

        @pl.when(i < NCHK // 2 - 1)
        def _():
            iload(c0 + 2, 0)

        @pl.when(i > 0)
        def _():
            swait(buf1, ssem1)

        iwait(1)
        compute(1, buf1)
        sstart(c0 + 1, buf1, ssem1)

        @pl.when(i < NCHK // 2 - 1)
        def _():
            iload(c0 + 3, 1)

        return carry

    lax.fori_loop(0, NCHK // 2, body, 0)
    swait(buf0, ssem0)
    swait(buf1, ssem1)


def kernel(points, batch_idx, imgs, intrinsics, extrinsics, W_dino):
    b, v, c, h, w = imgs.shape
    x = imgs.reshape(b * v, c, PATCH_GRID, 14, PATCH_GRID, 14)
    x = x.transpose(0, 2, 4, 1, 3, 5).reshape(b * v * PATCH_GRID * PATCH_GRID,
                                              c * 14 * 14)
    table = pl.pallas_call(
        _dino_matmul_kernel,
        out_shape=jax.ShapeDtypeStruct((TABLE_ROWS, DIM), jnp.float32),
    )(x, W_dino)
    table_pad = jnp.concatenate(
        [table, jnp.zeros((TPAD - TABLE_ROWS, DIM), jnp.float32)], axis=0)
    tblg = table_pad.reshape(TPAD, NCG, CW).transpose(1, 0, 2).reshape(-1)

    n = points.shape[0]
    pad = N_PAD - n
    pts = jnp.pad(points, ((0, pad), (0, 0)))
    bi = jnp.pad(batch_idx, (0, pad))
    xs = pts[:, 0].reshape(-1, P_COLS)
    ys = pts[:, 1].reshape(-1, P_COLS)
    zs = pts[:, 2].reshape(-1, P_COLS)
    bi2 = bi.reshape(-1, P_COLS)
    params = jnp.concatenate(
        [extrinsics.reshape(N_VIEWS, 12), intrinsics.reshape(N_VIEWS, 9),
         jnp.zeros((N_VIEWS, 3), jnp.float32)], axis=1)

    grid = N_PAD // P_BLK
    blk = pl.BlockSpec((P_ROWS, P_COLS), lambda i: (i, 0))
    idx = pl.pallas_call(
        _index_kernel,
        grid=(grid,),
        in_specs=[pl.BlockSpec(memory_space=pltpu.SMEM), blk, blk, blk, blk],
        out_specs=blk,
        out_shape=jax.ShapeDtypeStruct((N_PAD // P_COLS, P_COLS), jnp.int32),
    )(params, xs, ys, zs, bi2)

    mesh = plsc.VectorSubcoreMesh(core_axis_name="c", subcore_axis_name="s")
    tmp = pl.kernel(
        _gather_body,
        out_type=jax.ShapeDtypeStruct((NCG * N_PAD * CW,), jnp.float32),
        mesh=mesh,
        compiler_params=pltpu.CompilerParams(needs_layout_passes=False),
        scratch_types=[
            pltpu.VMEM((TPAD * CW,), jnp.float32),
            pltpu.VMEM((2, P), jnp.int32),
            pltpu.VMEM((P * CW,), jnp.float32),
            pltpu.VMEM((P * CW,), jnp.float32),
            pltpu.SemaphoreType.DMA,
            pltpu.SemaphoreType.DMA,
            pltpu.SemaphoreType.DMA,
            pltpu.SemaphoreType.DMA,
        ],
    )(tblg, idx.reshape(-1, P))
    return (tmp.reshape(NCG, N_PAD, CW).transpose(1, 0, 2)
            .reshape(N_PAD, DIM)[:n])


# speedup vs baseline: 2.2902x; 1.1698x over previous
import jax
import jax.numpy as jnp
from jax import lax
from jax.experimental import pallas as pl
from jax.experimental.pallas import tpu as pltpu
from jax.experimental.pallas import tpu_sc as plsc

DIM = 384
N_VIEWS = 8
PATCH_GRID = 16
TABLE_ROWS = N_VIEWS * PATCH_GRID * PATCH_GRID
TPAD = TABLE_ROWS + 8
SENTINEL = TABLE_ROWS

P_ROWS = 8
P_COLS = 256
P_BLK = P_ROWS * P_COLS
N_OUT = 100000
N_PAD = 100352

NCG = 8
CW = 48
NRG = 4
R_PER_G = N_OUT // NRG
P = 256
NCHK = 98
LOCAL_MAX = R_PER_G - P


def _dino_matmul_kernel(x_ref, w_ref, o_ref):
    o_ref[...] = jnp.dot(x_ref[...].astype(jnp.bfloat16),
                         w_ref[...].astype(jnp.bfloat16),
                         preferred_element_type=jnp.float32)


def _rb(t):
    return t.astype(jnp.bfloat16).astype(jnp.float32)


def _index_kernel(par_ref, x_ref, y_ref, z_ref, b_ref, o_ref):
    x = _rb(x_ref[...])
    y = _rb(y_ref[...])
    z = _rb(z_ref[...])
    bidx = b_ref[...]
    idx = jnp.full(x.shape, SENTINEL * CW, jnp.int32)
    for v8 in range(N_VIEWS):
        e = [_rb(par_ref[v8, i]) for i in range(12)]
        k = [_rb(par_ref[v8, 12 + i]) for i in range(9)]
        xc = e[0] * x + e[1] * y + e[2] * z + e[3]
        yc = e[4] * x + e[5] * y + e[6] * z + e[7]
        zc = e[8] * x + e[9] * y + e[10] * z + e[11]
        xcb, ycb, zcb = _rb(xc), _rb(yc), _rb(zc)
        xi = k[0] * xcb + k[1] * ycb + k[2] * zcb
        yi = k[3] * xcb + k[4] * ycb + k[5] * zcb
        zi = k[6] * xcb + k[7] * ycb + k[8] * zcb
        u = xi / zi
        v = yi / zi
        valid = ((zc > 0.1) & (u >= 0.0) & (u < 224.0)
                 & (v >= 0.0) & (v < 224.0) & (bidx == (v8 // 4)))
        up = jnp.clip((u / 14.0).astype(jnp.int32), 0, PATCH_GRID - 1)
        vp = jnp.clip((v / 14.0).astype(jnp.int32), 0, PATCH_GRID - 1)
        cand = (v8 * (PATCH_GRID * PATCH_GRID) + vp * PATCH_GRID + up) * CW
        idx = jnp.where(valid, cand, idx)
    o_ref[...] = idx


def _gather_body(tblg_hbm, idx_hbm, out_hbm, tbl_v, ibuf, buf0, buf1,
                 isem0, isem1, ssem0, ssem1):
    wid = lax.axis_index("s") * 2 + lax.axis_index("c")
    cg = wid // NRG
    rg = wid % NRG
    rbase = rg * R_PER_G
    tsz = TPAD * CW

    pltpu.sync_copy(tblg_hbm.at[pl.ds(cg * tsz, tsz)], tbl_v)

    siota = lax.iota(jnp.int32, 16) * CW

    def iload(c, par):
        local = jnp.minimum(c * P, LOCAL_MAX)
        sl = idx_hbm.at[pl.ds(rbase + local, P)]
        pltpu.async_copy(sl, ibuf.at[pl.ds(par * P, P)], (isem0, isem1)[par])

    def iwait(par):
        pltpu.make_async_copy(idx_hbm.at[pl.ds(0, P)],
                              ibuf.at[pl.ds(par * P, P)],
                              (isem0, isem1)[par]).wait()

    def compute(par, buf):
        def grp(q, carry):
            row48 = ibuf[pl.ds(par * P + q * 16, 16)]
            sb = siota + q * (16 * CW)
            for j0 in range(0, CW, 8):
                vals = [plsc.load_gather(tbl_v, [row48 + (j0 + t)])
                        for t in range(8)]
                for t in range(8):
                    plsc.store_scatter(buf, [sb + (j0 + t)], vals[t])
            return carry
        lax.fori_loop(0, P // 16, grp, 0)

    def sstart(c, buf, ssem):
        local = jnp.minimum(c * P, LOCAL_MAX)
        dst = out_hbm.at[pl.ds((cg * N_OUT + rbase + local) * CW, P * CW)]
        pltpu.async_copy(buf, dst, ssem)

    def swait(buf, ssem):
        pltpu.make_async_copy(
            buf, out_hbm.at[pl.ds(0, P * CW)], ssem).wait()

    iload(0, 0)
    iload(1, 1)

    def body(i, carry):
        c0 = 2 * i

        @pl.when(i > 0)
        def _():
            swait(buf0, ssem0)

        iwait(0)
        compute(0, buf0)
        sstart(c0, buf0, ssem0)

        @pl.when(i < NCHK // 2 - 1)
        def _():
            iload(c0 + 2, 0)

        @pl.when(i > 0)
        def _():
            swait(buf1, ssem1)

        iwait(1)
        compute(1, buf1)
        sstart(c0 + 1, buf1, ssem1)

        @pl.when(i < NCHK // 2 - 1)
        def _():
            iload(c0 + 3, 1)

        return carry

    lax.fori_loop(0, NCHK // 2, body, 0)
    swait(buf0, ssem0)
    swait(buf1, ssem1)


def kernel(points, batch_idx, imgs, intrinsics, extrinsics, W_dino):
    b, v, c, h, w = imgs.shape
    x = imgs.reshape(b * v, c, PATCH_GRID, 14, PATCH_GRID, 14)
    x = x.transpose(0, 2, 4, 1, 3, 5).reshape(b * v * PATCH_GRID * PATCH_GRID,
                                              c * 14 * 14)
    table = pl.pallas_call(
        _dino_matmul_kernel,
        out_shape=jax.ShapeDtypeStruct((TABLE_ROWS, DIM), jnp.float32),
    )(x, W_dino)
    table_pad = jnp.concatenate(
        [table, jnp.zeros((TPAD - TABLE_ROWS, DIM), jnp.float32)], axis=0)
    tblg = table_pad.reshape(TPAD, NCG, CW).transpose(1, 0, 2).reshape(-1)

    n = points.shape[0]
    pad = N_PAD - n
    pts = jnp.pad(points, ((0, pad), (0, 0)))
    bi = jnp.pad(batch_idx, (0, pad))
    xs = pts[:, 0].reshape(-1, P_COLS)
    ys = pts[:, 1].reshape(-1, P_COLS)
    zs = pts[:, 2].reshape(-1, P_COLS)
    bi2 = bi.reshape(-1, P_COLS)
    params = jnp.concatenate(
        [extrinsics.reshape(N_VIEWS, 12), intrinsics.reshape(N_VIEWS, 9),
         jnp.zeros((N_VIEWS, 3), jnp.float32)], axis=1)

    grid = N_PAD // P_BLK
    blk = pl.BlockSpec((P_ROWS, P_COLS), lambda i: (i, 0))
    idx = pl.pallas_call(
        _index_kernel,
        grid=(grid,),
        in_specs=[pl.BlockSpec(memory_space=pltpu.SMEM), blk, blk, blk, blk],
        out_specs=blk,
        out_shape=jax.ShapeDtypeStruct((N_PAD // P_COLS, P_COLS), jnp.int32),
    )(params, xs, ys, zs, bi2)

    mesh = plsc.VectorSubcoreMesh(core_axis_name="c", subcore_axis_name="s")
    tmp = pl.kernel(
        _gather_body,
        out_type=jax.ShapeDtypeStruct((NCG * N_OUT * CW,), jnp.float32),
        mesh=mesh,
        compiler_params=pltpu.CompilerParams(needs_layout_passes=False),
        scratch_types=[
            pltpu.VMEM((TPAD * CW,), jnp.float32),
            pltpu.VMEM((2 * P,), jnp.int32),
            pltpu.VMEM((P * CW,), jnp.float32),
            pltpu.VMEM((P * CW,), jnp.float32),
            pltpu.SemaphoreType.DMA,
            pltpu.SemaphoreType.DMA,
            pltpu.SemaphoreType.DMA,
            pltpu.SemaphoreType.DMA,
        ],
    )(tblg, idx.reshape(-1))
    return tmp.reshape(NCG, N_OUT, CW).transpose(1, 0, 2).reshape(N_OUT, DIM)


# speedup vs baseline: 2.3227x; 1.0142x over previous
import jax
import jax.numpy as jnp
from jax import lax
from jax.experimental import pallas as pl
from jax.experimental.pallas import tpu as pltpu
from jax.experimental.pallas import tpu_sc as plsc

DIM = 384
N_VIEWS = 8
PATCH_GRID = 16
TABLE_ROWS = N_VIEWS * PATCH_GRID * PATCH_GRID
TPAD = TABLE_ROWS + 8
SENTINEL = TABLE_ROWS

P_ROWS = 8
P_COLS = 256
P_BLK = P_ROWS * P_COLS
N_OUT = 100000
N_PAD = 100352

NCG = 8
CW = 48
NRG = 4
R_PER_G = N_OUT // NRG
P = 256
NCHK = 98
LOCAL_MAX = R_PER_G - P


def _dino_matmul_kernel(x_ref, w_ref, o_ref):
    o_ref[...] = jnp.dot(x_ref[...].astype(jnp.bfloat16),
                         w_ref[...].astype(jnp.bfloat16),
                         preferred_element_type=jnp.float32)


def _rb(t):
    return t.astype(jnp.bfloat16).astype(jnp.float32)


def _index_kernel(par_ref, x_ref, y_ref, z_ref, b_ref, o_ref):
    x = _rb(x_ref[...])
    y = _rb(y_ref[...])
    z = _rb(z_ref[...])
    bidx = b_ref[...]
    idx = jnp.full(x.shape, SENTINEL * CW, jnp.int32)
    for v8 in range(N_VIEWS):
        e = [_rb(par_ref[v8, i]) for i in range(12)]
        k = [_rb(par_ref[v8, 12 + i]) for i in range(9)]
        xc = e[0] * x + e[1] * y + e[2] * z + e[3]
        yc = e[4] * x + e[5] * y + e[6] * z + e[7]
        zc = e[8] * x + e[9] * y + e[10] * z + e[11]
        xcb, ycb, zcb = _rb(xc), _rb(yc), _rb(zc)
        xi = k[0] * xcb + k[1] * ycb + k[2] * zcb
        yi = k[3] * xcb + k[4] * ycb + k[5] * zcb
        zi = k[6] * xcb + k[7] * ycb + k[8] * zcb
        u = xi / zi
        v = yi / zi
        valid = ((zc > 0.1) & (u >= 0.0) & (u < 224.0)
                 & (v >= 0.0) & (v < 224.0) & (bidx == (v8 // 4)))
        up = jnp.clip((u / 14.0).astype(jnp.int32), 0, PATCH_GRID - 1)
        vp = jnp.clip((v / 14.0).astype(jnp.int32), 0, PATCH_GRID - 1)
        cand = (v8 * (PATCH_GRID * PATCH_GRID) + vp * PATCH_GRID + up) * CW
        idx = jnp.where(valid, cand, idx)
    o_ref[...] = idx


def _gather_body(tblg_hbm, idx_hbm, out_hbm, tbl_v, ibuf, buf0, buf1,
                 isem0, isem1, ssem0, ssem1):
    wid = lax.axis_index("s") * 2 + lax.axis_index("c")
    cg = wid // NRG
    rg = wid % NRG
    rbase = rg * R_PER_G
    tsz = TPAD * CW

    pltpu.sync_copy(tblg_hbm.at[pl.ds(cg * tsz, tsz)], tbl_v)

    siota = lax.iota(jnp.int32, 16) * CW

    def iload(c, par):
        local = jnp.minimum(c * P, LOCAL_MAX)
        sl = idx_hbm.at[pl.ds(rbase + local, P)]
        pltpu.async_copy(sl, ibuf.at[pl.ds(par * P, P)], (isem0, isem1)[par])

    def iwait(par):
        pltpu.make_async_copy(idx_hbm.at[pl.ds(0, P)],
                              ibuf.at[pl.ds(par * P, P)],
                              (isem0, isem1)[par]).wait()

    def compute(par, buf):
        def grp(q2, carry):
            for h in range(2):
                q = q2 * 2 + h
                row48 = ibuf[pl.ds(par * P + q * 16, 16)]
                sb = siota + q * (16 * CW)
                for j0 in range(0, CW, 12):
                    vals = [plsc.load_gather(tbl_v, [row48 + (j0 + t)])
                            for t in range(12)]
                    for t in range(12):
                        plsc.store_scatter(buf, [sb + (j0 + t)], vals[t])
            return carry
        lax.fori_loop(0, P // 32, grp, 0)

    def sstart(c, buf, ssem):
        local = jnp.minimum(c * P, LOCAL_MAX)
        dst = out_hbm.at[pl.ds((cg * N_OUT + rbase + local) * CW, P * CW)]
        pltpu.async_copy(buf, dst, ssem)

    def swait(buf, ssem):
        pltpu.make_async_copy(
            buf, out_hbm.at[pl.ds(0, P * CW)], ssem).wait()

    iload(0, 0)
    iload(1, 1)

    def body(i, carry):
        c0 = 2 * i

        @pl.when(i > 0)
        def _():
            swait(buf0, ssem0)

        iwait(0)
        compute(0, buf0)
        sstart(c0, buf0, ssem0)

        @pl.when(i < NCHK // 2 - 1)
        def _():
            iload(c0 + 2, 0)

        @pl.when(i > 0)
        def _():
            swait(buf1, ssem1)

        iwait(1)
        compute(1, buf1)
        sstart(c0 + 1, buf1, ssem1)

        @pl.when(i < NCHK // 2 - 1)
        def _():
            iload(c0 + 3, 1)

        return carry

    lax.fori_loop(0, NCHK // 2, body, 0)
    swait(buf0, ssem0)
    swait(buf1, ssem1)


def kernel(points, batch_idx, imgs, intrinsics, extrinsics, W_dino):
    b, v, c, h, w = imgs.shape
    x = imgs.reshape(b * v, c, PATCH_GRID, 14, PATCH_GRID, 14)
    x = x.transpose(0, 2, 4, 1, 3, 5).reshape(b * v * PATCH_GRID * PATCH_GRID,
                                              c * 14 * 14)
    table = pl.pallas_call(
        _dino_matmul_kernel,
        out_shape=jax.ShapeDtypeStruct((TABLE_ROWS, DIM), jnp.float32),
    )(x, W_dino)
    table_pad = jnp.concatenate(
        [table, jnp.zeros((TPAD - TABLE_ROWS, DIM), jnp.float32)], axis=0)
    tblg = table_pad.reshape(TPAD, NCG, CW).transpose(1, 0, 2).reshape(-1)

    n = points.shape[0]
    pad = N_PAD - n
    pts = jnp.pad(points, ((0, pad), (0, 0)))
    bi = jnp.pad(batch_idx, (0, pad))
    xs = pts[:, 0].reshape(-1, P_COLS)
    ys = pts[:, 1].reshape(-1, P_COLS)
    zs = pts[:, 2].reshape(-1, P_COLS)
    bi2 = bi.reshape(-1, P_COLS)
    params = jnp.concatenate(
        [extrinsics.reshape(N_VIEWS, 12), intrinsics.reshape(N_VIEWS, 9),
         jnp.zeros((N_VIEWS, 3), jnp.float32)], axis=1)

    grid = N_PAD // P_BLK
    blk = pl.BlockSpec((P_ROWS, P_COLS), lambda i: (i, 0))
    idx = pl.pallas_call(
        _index_kernel,
        grid=(grid,),
        in_specs=[pl.BlockSpec(memory_space=pltpu.SMEM), blk, blk, blk, blk],
        out_specs=blk,
        out_shape=jax.ShapeDtypeStruct((N_PAD // P_COLS, P_COLS), jnp.int32),
    )(params, xs, ys, zs, bi2)

    mesh = plsc.VectorSubcoreMesh(core_axis_name="c", subcore_axis_name="s")
    tmp = pl.kernel(
        _gather_body,
        out_type=jax.ShapeDtypeStruct((NCG * N_OUT * CW,), jnp.float32),
        mesh=mesh,
        compiler_params=pltpu.CompilerParams(needs_layout_passes=False),
        scratch_types=[
            pltpu.VMEM((TPAD * CW,), jnp.float32),
            pltpu.VMEM((2 * P,), jnp.int32),
            pltpu.VMEM((P * CW,), jnp.float32),
            pltpu.VMEM((P * CW,), jnp.float32),
            pltpu.SemaphoreType.DMA,
            pltpu.SemaphoreType.DMA,
            pltpu.SemaphoreType.DMA,
            pltpu.SemaphoreType.DMA,
        ],
    )(tblg, idx.reshape(-1))
    return tmp.reshape(NCG, N_OUT, CW).transpose(1, 0, 2).reshape(N_OUT, DIM)
